# baseline (device time: 12089 ns/iter reference)
import functools

import jax
import jax.numpy as jnp
from jax import lax
from jax.experimental import pallas as pl
from jax.experimental.pallas import tpu as pltpu

N_DEV = 8
N_TOK = 256
D_IN = 128
D_OUT = 256
N_EXP = 16
EXP_PER_DEV = N_EXP // N_DEV
ROWS = N_TOK // N_DEV


def kernel(x, router_W, route_idx, expert_W, shared_W):
    def body(x_ref, rw_ref, idx_ref, ew_ref, sw_ref, out_ref,
             acc_ref, recv_ref, send_sems, recv_sems):
        my = lax.axis_index("i")

        barrier_sem = pltpu.get_barrier_semaphore()
        for k in range(1, N_DEV):
            pl.semaphore_signal(
                barrier_sem, inc=1,
                device_id=((my + k) % N_DEV,),
                device_id_type=pl.DeviceIdType.MESH,
            )
        pl.semaphore_wait(barrier_sem, N_DEV - 1)

        scores = jnp.dot(x_ref[:, :], rw_ref[:, :],
                         preferred_element_type=jnp.float32)
        s_max = jnp.max(scores, axis=1, keepdims=True)
        e_s = jnp.exp(scores - s_max)
        probs = e_s / jnp.sum(e_s, axis=1, keepdims=True)

        idx = idx_ref[:, :]
        onehot = idx == lax.broadcasted_iota(jnp.int32, (N_TOK, N_EXP), 1)
        p_route = jnp.sum(jnp.where(onehot, probs, 0.0),
                          axis=1, keepdims=True)

        xb = x_ref[:, :].astype(jnp.bfloat16)
        e0 = my * EXP_PER_DEV
        acc = jnp.zeros((N_TOK, D_OUT), jnp.float32)
        for j in range(EXP_PER_DEV):
            w = jnp.where(idx == e0 + j, p_route, 0.0)
            y = jnp.dot(xb, ew_ref[j].astype(jnp.bfloat16),
                        preferred_element_type=jnp.float32)
            acc = acc + w * y
        acc_ref[:, :] = acc

        sends = []
        for k in range(1, N_DEV):
            dst = (my + k) % N_DEV
            rdma = pltpu.make_async_remote_copy(
                src_ref=acc_ref.at[pl.ds(dst * ROWS, ROWS), :],
                dst_ref=recv_ref.at[my],
                send_sem=send_sems.at[k],
                recv_sem=recv_sems.at[my],
                device_id=(dst,),
                device_id_type=pl.DeviceIdType.MESH,
            )
            rdma.start()
            sends.append(rdma)

        recv_ref[pl.ds(my, 1), :, :] = acc_ref[pl.ds(my * ROWS, ROWS), :][None]
        xm = x_ref[pl.ds(my * ROWS, ROWS), :].astype(jnp.bfloat16)
        shared = jnp.dot(xm, sw_ref[:, :].astype(jnp.bfloat16),
                         preferred_element_type=jnp.float32)

        for k in range(1, N_DEV):
            src = (my + k) % N_DEV
            recv = pltpu.make_async_remote_copy(
                src_ref=acc_ref.at[pl.ds(0, ROWS), :],
                dst_ref=recv_ref.at[src],
                send_sem=send_sems.at[0],
                recv_sem=recv_sems.at[src],
                device_id=(my,),
                device_id_type=pl.DeviceIdType.MESH,
            )
            recv.wait_recv()

        out_ref[:, :] = shared + jnp.sum(recv_ref[:, :, :], axis=0)

        for rdma in sends:
            rdma.wait_send()

        @functools.partial(pl.run_scoped, sem=pltpu.SemaphoreType.REGULAR)
        def _(sem):
            for k in range(1, N_DEV):
                pl.semaphore_signal(
                    sem, inc=1,
                    device_id=((my + k) % N_DEV,),
                    device_id_type=pl.DeviceIdType.MESH,
                )
            pl.semaphore_wait(sem, N_DEV - 1)

    return pl.pallas_call(
        body,
        out_shape=jax.ShapeDtypeStruct((ROWS, D_OUT), jnp.float32),
        in_specs=[pl.BlockSpec(memory_space=pltpu.VMEM)] * 5,
        out_specs=pl.BlockSpec(memory_space=pltpu.VMEM),
        scratch_shapes=[
            pltpu.VMEM((N_TOK, D_OUT), jnp.float32),
            pltpu.VMEM((N_DEV, ROWS, D_OUT), jnp.float32),
            pltpu.SemaphoreType.DMA((N_DEV,)),
            pltpu.SemaphoreType.DMA((N_DEV,)),
        ],
        compiler_params=pltpu.CompilerParams(collective_id=0),
    )(x, router_W, route_idx, expert_W, shared_W)
